# baseline (speedup 1.0000x reference)
import jax
import jax.numpy as jnp
from jax.experimental import pallas as pl

_ROW_BLOCK = 400


def _fused_heads_kernel(x_ref, w_ref, b_ref, o_ref):
    o_ref[...] = (
        jnp.dot(x_ref[...], w_ref[...], preferred_element_type=jnp.float32)
        + b_ref[...]
    )


def kernel(x, W_cls, b_cls, W_box, b_box):
    if x.ndim > 2:
        x = x.reshape(x.shape[0], -1)
    R, D = x.shape
    n_cls = W_cls.shape[1]
    n_all = n_cls + W_box.shape[1]
    cp = max(128, ((n_all + 127) // 128) * 128)

    W = jnp.concatenate([W_cls, W_box], axis=1)
    W = jnp.pad(W, ((0, 0), (0, cp - n_all)))
    b = jnp.pad(jnp.concatenate([b_cls, b_box]), (0, cp - n_all)).reshape(1, cp)

    out = pl.pallas_call(
        _fused_heads_kernel,
        grid=(pl.cdiv(R, _ROW_BLOCK),),
        in_specs=[
            pl.BlockSpec((_ROW_BLOCK, D), lambda i: (i, 0)),
            pl.BlockSpec((D, cp), lambda i: (0, 0)),
            pl.BlockSpec((1, cp), lambda i: (0, 0)),
        ],
        out_specs=pl.BlockSpec((_ROW_BLOCK, cp), lambda i: (i, 0)),
        out_shape=jax.ShapeDtypeStruct((R, cp), jnp.float32),
    )(x, W, b)

    return out[:, :n_cls], out[:, n_cls:n_all]


# speedup vs baseline: 1.1918x; 1.1918x over previous
import jax
import jax.numpy as jnp
from jax.experimental import pallas as pl

_ROW_BLOCK = 1000


def _fused_heads_kernel(x_ref, w_ref, b_ref, o1_ref, o2_ref):
    acc = b_ref[...] + jnp.dot(x_ref[...], w_ref[...],
                               preferred_element_type=jnp.float32)
    o1_ref[...] = acc[:, : o1_ref.shape[1]]
    o2_ref[...] = acc[:, 128 : 128 + o2_ref.shape[1]]


def kernel(x, W_cls, b_cls, W_box, b_box):
    if x.ndim > 2:
        x = x.reshape(x.shape[0], -1)
    R, D = x.shape
    n1 = W_cls.shape[1]
    n2 = W_box.shape[1]

    W = jnp.concatenate(
        [jnp.pad(W_cls, ((0, 0), (0, 128 - n1))),
         jnp.pad(W_box, ((0, 0), (0, 128 - n2)))], axis=1)
    b = jnp.concatenate(
        [jnp.pad(b_cls, (0, 128 - n1)), jnp.pad(b_box, (0, 128 - n2))]
    ).reshape(1, 256)

    o1, o2 = pl.pallas_call(
        _fused_heads_kernel,
        grid=(pl.cdiv(R, _ROW_BLOCK),),
        in_specs=[
            pl.BlockSpec((_ROW_BLOCK, D), lambda i: (i, 0)),
            pl.BlockSpec((D, 256), lambda i: (0, 0)),
            pl.BlockSpec((1, 256), lambda i: (0, 0)),
        ],
        out_specs=[
            pl.BlockSpec((_ROW_BLOCK, n1), lambda i: (i, 0)),
            pl.BlockSpec((_ROW_BLOCK, n2), lambda i: (i, 0)),
        ],
        out_shape=[
            jax.ShapeDtypeStruct((R, n1), jnp.float32),
            jax.ShapeDtypeStruct((R, n2), jnp.float32),
        ],
    )(x, W, b)

    return o1, o2
